# baseline (speedup 1.0000x reference)
import functools

import jax
import jax.numpy as jnp
from jax import lax
from jax.experimental import pallas as pl
from jax.experimental.pallas import tpu as pltpu
from jax.experimental.pallas import tpu_sc as plsc

B = 16384
S = 50
D = 64
TAB = 64
NC = 2
NS = 16
NW = NC * NS
RPW = B // NW
G = 16
NG = RPW // G
GP = G * S
GE = GP * D


def _build():
    mesh = plsc.VectorSubcoreMesh(core_axis_name="c", subcore_axis_name="s")

    @functools.partial(
        pl.kernel,
        mesh=mesh,
        compiler_params=pltpu.CompilerParams(
            needs_layout_passes=False, use_tc_tiling_on_sc=False),
        out_type=jax.ShapeDtypeStruct((B * S * D,), jnp.float32),
        scratch_types=[
            pltpu.VMEM((TAB * D,), jnp.float32),
            pltpu.VMEM((2, GP), jnp.int32),
            pltpu.VMEM((2, GE), jnp.float32),
            pltpu.SemaphoreType.DMA,
            pltpu.SemaphoreType.DMA,
            pltpu.SemaphoreType.DMA,
            pltpu.SemaphoreType.DMA,
        ],
    )
    def emb_kernel(in_hbm, w_hbm, out_hbm, tab_ref, tok_ref, out_ref,
                   tsem0, tsem1, wsem0, wsem1):
        wid = lax.axis_index("s") * NC + lax.axis_index("c")
        flat0 = wid * RPW * S

        pltpu.sync_copy(w_hbm.at[pl.ds(0, TAB * D)], tab_ref)

        iota = lax.iota(jnp.int32, 16)
        lane_tok = iota * S
        lane_out = iota * (S * D)

        tsems = (tsem0, tsem1)
        wsems = (wsem0, wsem1)

        def tok_fetch(g, slot):
            return pltpu.make_async_copy(
                in_hbm.at[pl.ds(flat0 + g * GP, GP)], tok_ref.at[slot],
                tsems[slot])

        def out_write(g, slot):
            return pltpu.make_async_copy(
                out_ref.at[slot],
                out_hbm.at[pl.ds((flat0 + g * GP) * D, GE)], wsems[slot])

        tok_fetch(0, 0).start()

        def group_body(i, carry):
            for b in range(2):
                g = 2 * i + b
                ob = 1 - b
                @pl.when(g + 1 < NG)
                def _():
                    tok_fetch(g + 1, ob).start()
                tok_fetch(g, b).wait()
                @pl.when(g >= 2)
                def _():
                    out_write(g - 2, b).wait()

                def step(s, running):
                    tok = plsc.load_gather(tok_ref.at[b], [lane_tok + s])
                    m = tok != 0
                    running = running + m.astype(jnp.int32)
                    pos = jnp.where(m, running, 0)
                    src = pos * D
                    dst = lane_out + s * D
                    for j in range(D):
                        val = plsc.load_gather(tab_ref, [src + j])
                        plsc.store_scatter(out_ref.at[b], [dst + j], val)
                    return running

                lax.fori_loop(0, S, step, jnp.zeros((16,), jnp.int32))
                out_write(g, b).start()
            return carry

        lax.fori_loop(0, NG // 2, group_body, 0)

        out_write(NG - 2, 0).wait()
        out_write(NG - 1, 1).wait()

    return emb_kernel


_EMB = _build()


@jax.jit
def kernel(input, weights):
    out = _EMB(input.reshape(B * S), weights.reshape(1024 * D))
    return out.reshape(B, S, D)


# speedup vs baseline: 2.2033x; 2.2033x over previous
import functools

import jax
import jax.numpy as jnp
from jax import lax
from jax.experimental import pallas as pl
from jax.experimental.pallas import tpu as pltpu
from jax.experimental.pallas import tpu_sc as plsc

B = 16384
S = 50
D = 64
TAB = 64
NC = 2
NS = 16
NW = NC * NS
RPW = B // NW
G = 16
NG = RPW // G
GP = G * S
GE = GP * D


def _build():
    mesh = plsc.VectorSubcoreMesh(core_axis_name="c", subcore_axis_name="s")

    @functools.partial(
        pl.kernel,
        mesh=mesh,
        compiler_params=pltpu.CompilerParams(
            needs_layout_passes=False, use_tc_tiling_on_sc=False,
            disable_bounds_checks=True),
        out_type=jax.ShapeDtypeStruct((B * S * D,), jnp.float32),
        scratch_types=[
            pltpu.VMEM((TAB * D,), jnp.float32),
            pltpu.VMEM((2, GP), jnp.int32),
            pltpu.VMEM((2, GE), jnp.float32),
            pltpu.SemaphoreType.DMA,
            pltpu.SemaphoreType.DMA,
            pltpu.SemaphoreType.DMA,
            pltpu.SemaphoreType.DMA,
        ],
    )
    def emb_kernel(in_hbm, w_hbm, out_hbm, tab_ref, tok_ref, out_ref,
                   tsem0, tsem1, wsem0, wsem1):
        wid = lax.axis_index("s") * NC + lax.axis_index("c")
        flat0 = wid * RPW * S

        pltpu.sync_copy(w_hbm.at[pl.ds(0, TAB * D)], tab_ref)

        iota = lax.iota(jnp.int32, 16)
        lane_tok = iota * S
        lane_out = iota * (S * D)

        tsems = (tsem0, tsem1)
        wsems = (wsem0, wsem1)

        def tok_fetch(g, slot):
            return pltpu.make_async_copy(
                in_hbm.at[pl.ds(flat0 + g * GP, GP)], tok_ref.at[slot],
                tsems[slot])

        def out_write(g, slot):
            return pltpu.make_async_copy(
                out_ref.at[slot],
                out_hbm.at[pl.ds((flat0 + g * GP) * D, GE)], wsems[slot])

        tok_fetch(0, 0).start()

        def group_body(i, carry):
            for b in range(2):
                g = 2 * i + b
                ob = 1 - b
                @pl.when(g + 1 < NG)
                def _():
                    tok_fetch(g + 1, ob).start()
                tok_fetch(g, b).wait()
                @pl.when(g >= 2)
                def _():
                    out_write(g - 2, b).wait()

                def step(s, running):
                    tok = plsc.load_gather(tok_ref.at[b], [lane_tok + s])
                    m = tok != 0
                    running = running + m.astype(jnp.int32)
                    pos = jnp.where(m, running, 0)
                    src = pos * D
                    dst = lane_out + s * D
                    lag = 8
                    vals = {}
                    for j in range(D + lag):
                        if j < D:
                            c = jnp.bitwise_xor(iota, j)
                            vals[j] = (c, plsc.load_gather(tab_ref, [src + c]))
                        if j >= lag:
                            c, v = vals.pop(j - lag)
                            plsc.store_scatter(out_ref.at[b], [dst + c], v)
                    return running

                lax.fori_loop(0, S, step, jnp.zeros((16,), jnp.int32))
                out_write(g, b).start()
            return carry

        lax.fori_loop(0, NG // 2, group_body, 0)

        out_write(NG - 2, 0).wait()
        out_write(NG - 1, 1).wait()

    return emb_kernel


_EMB = _build()


@jax.jit
def kernel(input, weights):
    out = _EMB(input.reshape(B * S), weights.reshape(1024 * D))
    return out.reshape(B, S, D)


# speedup vs baseline: 3.0972x; 1.4057x over previous
import functools

import jax
import jax.numpy as jnp
from jax import lax
from jax.experimental import pallas as pl
from jax.experimental.pallas import tpu as pltpu
from jax.experimental.pallas import tpu_sc as plsc

B = 16384
S = 50
D = 64
TAB = 64
NC = 2
NS = 16
NW = NC * NS
RPW = B // NW
G = 16
NG = RPW // G
GP = G * S
GE = GP * D


def _build():
    mesh = plsc.VectorSubcoreMesh(core_axis_name="c", subcore_axis_name="s")

    @functools.partial(
        pl.kernel,
        mesh=mesh,
        compiler_params=pltpu.CompilerParams(
            needs_layout_passes=False, use_tc_tiling_on_sc=False,
            disable_bounds_checks=True),
        out_type=jax.ShapeDtypeStruct((B * S * D,), jnp.float32),
        scratch_types=[
            pltpu.VMEM((TAB * D,), jnp.float32),
            pltpu.VMEM((2, GP), jnp.int32),
            pltpu.VMEM((2, GE), jnp.float32),
            pltpu.SemaphoreType.DMA,
            pltpu.SemaphoreType.DMA,
            pltpu.SemaphoreType.DMA,
            pltpu.SemaphoreType.DMA,
        ],
    )
    def emb_kernel(in_hbm, w_hbm, out_hbm, tab_ref, tok_ref, out_ref,
                   tsem0, tsem1, wsem0, wsem1):
        wid = lax.axis_index("s") * NC + lax.axis_index("c")
        flat0 = wid * RPW * S

        pltpu.sync_copy(w_hbm.at[pl.ds(0, TAB * D)], tab_ref)

        iota = lax.iota(jnp.int32, 16)
        lane_tok = iota * S
        lane_out = iota * (S * D)

        tsems = (tsem0, tsem1)
        wsems = (wsem0, wsem1)

        def tok_fetch(g, slot):
            return pltpu.make_async_copy(
                in_hbm.at[pl.ds(flat0 + g * GP, GP)], tok_ref.at[slot],
                tsems[slot])

        def out_write(g, slot):
            return pltpu.make_async_copy(
                out_ref.at[slot],
                out_hbm.at[pl.ds((flat0 + g * GP) * D, GE)], wsems[slot])

        tok_fetch(0, 0).start()

        def group_body(i, carry):
            for b in range(2):
                g = 2 * i + b
                ob = 1 - b
                @pl.when(g + 1 < NG)
                def _():
                    tok_fetch(g + 1, ob).start()
                tok_fetch(g, b).wait()
                @pl.when(g >= 2)
                def _():
                    out_write(g - 2, b).wait()

                def step(s, running):
                    tok = plsc.load_gather(tok_ref.at[b], [lane_tok + s])
                    m = tok != 0
                    running = running + m.astype(jnp.int32)
                    pos = jnp.where(m, running, 0)
                    src = pos * D + iota
                    dst = lane_out + s * D + iota
                    lag = 8
                    vals = {}
                    for j in range(D + lag):
                        if j < D:
                            vals[j] = (dst, plsc.load_gather(tab_ref, [src]))
                            if j + 1 < D:
                                bit = (j + 1) & -(j + 1)
                                src = jnp.bitwise_xor(src, bit)
                                dst = jnp.bitwise_xor(dst, bit)
                        if j >= lag:
                            d, v = vals.pop(j - lag)
                            plsc.store_scatter(out_ref.at[b], [d], v)
                    return running

                lax.fori_loop(0, S, step, jnp.zeros((16,), jnp.int32))
                out_write(g, b).start()
            return carry

        lax.fori_loop(0, NG // 2, group_body, 0)

        out_write(NG - 2, 0).wait()
        out_write(NG - 1, 1).wait()

    return emb_kernel


_EMB = _build()


@jax.jit
def kernel(input, weights):
    out = _EMB(input.reshape(B * S), weights.reshape(1024 * D))
    return out.reshape(B, S, D)
